# Initial kernel scaffold; baseline (speedup 1.0000x reference)
import jax
import jax.numpy as jnp
from jax.experimental import pallas as pl


def kernel(x, edge_index):
    raise NotImplementedError("write your pallas kernel here")


# speedup vs baseline: 2.8024x; 2.8024x over previous
import functools

import jax
import jax.numpy as jnp
from jax import lax
from jax.experimental import pallas as pl
from jax.experimental.pallas import tpu as pltpu
from jax.experimental.pallas import tpu_sc as plsc

E = 320000
D = 128
NW = 32
PER_W = E // NW
C = 80
NCH = PER_W // C

_mesh = plsc.VectorSubcoreMesh(core_axis_name="c", subcore_axis_name="s")


@functools.partial(
    pl.kernel,
    mesh=_mesh,
    out_type=jax.ShapeDtypeStruct((E,), jnp.float32),
    scratch_types=[
        pltpu.VMEM((C,), jnp.int32),
        pltpu.VMEM((C,), jnp.int32),
        pltpu.VMEM((C, D), jnp.float32),
        pltpu.VMEM((C, D), jnp.float32),
        pltpu.VMEM((C,), jnp.float32),
        pltpu.SemaphoreType.DMA,
    ],
    compiler_params=pltpu.CompilerParams(needs_layout_passes=False),
)
def _edge_dot(x_hbm, src_hbm, dst_hbm, out_hbm, src_v, dst_v, u_v, v_v, o_v, sem):
    wid = lax.axis_index("c") * 16 + lax.axis_index("s")
    base = wid * PER_W
    lane = lax.iota(jnp.int32, 16)

    def chunk_body(k, carry):
        off = base + k * jnp.int32(C)
        pltpu.sync_copy(src_hbm.at[pl.ds(off, C)], src_v)
        pltpu.sync_copy(dst_hbm.at[pl.ds(off, C)], dst_v)
        cp_u = pltpu.async_copy(x_hbm.at[src_v], u_v, sem)
        cp_v = pltpu.async_copy(x_hbm.at[dst_v], v_v, sem)
        cp_u.wait()
        cp_v.wait()

        def group_body(g, carry2):
            e0 = g * jnp.int32(16)
            acc = jnp.zeros((16,), jnp.float32)
            for j in range(16):
                e = e0 + jnp.int32(j)
                parts = [
                    u_v[e, pl.ds(kk * 16, 16)] * v_v[e, pl.ds(kk * 16, 16)]
                    for kk in range(D // 16)
                ]
                s0 = (parts[0] + parts[1]) + (parts[2] + parts[3])
                s1 = (parts[4] + parts[5]) + (parts[6] + parts[7])
                s = jnp.sum(s0 + s1)
                acc = jnp.where(lane == jnp.int32(j), s, acc)
            o_v[pl.ds(e0, 16)] = acc
            return carry2

        lax.fori_loop(jnp.int32(0), jnp.int32(C // 16), group_body, jnp.int32(0))
        pltpu.sync_copy(o_v, out_hbm.at[pl.ds(off, C)])
        return carry

    lax.fori_loop(jnp.int32(0), jnp.int32(NCH), chunk_body, jnp.int32(0))


def kernel(x, edge_index):
    ei = edge_index.astype(jnp.int32)
    return _edge_dot(x, ei[0], ei[1])
